# baseline (speedup 1.0000x reference)
import functools

import jax
import jax.numpy as jnp
from jax import lax
from jax.experimental import pallas as pl
from jax.experimental.pallas import tpu as pltpu
import jax.experimental.pallas.tpu_sc as plsc

N_NODES = 10000
N_FEAT = 128
N_HID = 16
N_HEADS = 8
N_CLASSES = 16
N_EDGES = 320000

NC = 2
NS = 16
NW = NC * NS
E_PER_W = N_EDGES // NW
CHUNK = 80
N_CHUNKS = E_PER_W // CHUNK
N_PAD = 10240
ROWS_PER_TILE = N_PAD // NS

F1 = 144
F1P = 80
F2 = 32

NRB = 2
NIB = 4
UNROLL = 4


def _leaky(a):
    return jnp.where(a >= 0, a, 0.2 * a)


def _tc_prep_body(x_ref, w1_ref, asrc_ref, adst_ref, t1s_ref, t1p_ref, t1d_ref):
    h = jnp.dot(x_ref[...], w1_ref[...], preferred_element_type=jnp.float32)
    a_s = jnp.dot(h, asrc_ref[...], preferred_element_type=jnp.float32)
    a_d = jnp.dot(h, adst_ref[...], preferred_element_type=jnp.float32)
    pad8 = jnp.zeros((N_NODES, 8), jnp.float32)
    t1s = jnp.concatenate([h, a_s, pad8], axis=1)
    t1s_ref[...] = t1s
    h160 = jnp.concatenate([t1s, jnp.zeros((N_NODES, 16), jnp.float32)], axis=1)
    lo = jnp.concatenate([h160[:, 32 * j:32 * j + 16] for j in range(5)], axis=1)
    hi = jnp.concatenate([h160[:, 32 * j + 16:32 * j + 32] for j in range(5)], axis=1)
    lo_u = lax.bitcast_convert_type(
        lo.astype(jnp.bfloat16).astype(jnp.float32), jnp.uint32) >> 16
    hi_u = lax.bitcast_convert_type(
        hi.astype(jnp.bfloat16).astype(jnp.float32), jnp.uint32) & jnp.uint32(0xFFFF0000)
    t1p_ref[...] = lax.bitcast_convert_type(lo_u | hi_u, jnp.int32)
    t1d_ref[...] = jnp.concatenate(
        [jnp.concatenate([a_d, pad8], axis=1),
         jnp.zeros((N_PAD - N_NODES, 16), jnp.float32)], axis=0)


def _tc_combine1_body(p0_ref, p1_ref, t1s_ref, t1d_ref, b1_ref, w2_ref,
                      a2s_ref, a2d_ref, rep_ref, t2s_ref, t2d_ref):
    P = p0_ref[:N_NODES] + p1_ref[:N_NODES]
    msg = P[:, :N_FEAT]
    den = P[:, N_FEAT:N_FEAT + N_HEADS]
    h1 = t1s_ref[:, :N_FEAT]
    as1 = t1s_ref[:, N_FEAT:N_FEAT + N_HEADS]
    ad1 = t1d_ref[:N_NODES, :N_HEADS]
    wself = jnp.exp(_leaky(as1 + ad1))
    rep = rep_ref[...]
    num = msg + jnp.dot(wself, rep, preferred_element_type=jnp.float32) * h1
    den_w = jnp.dot(den + wself, rep, preferred_element_type=jnp.float32)
    out1 = num / (den_w + 1e-16) + b1_ref[...]
    hidden = jnp.where(out1 > 0, out1, jnp.exp(out1) - 1.0)
    h2 = jnp.dot(hidden, w2_ref[...], preferred_element_type=jnp.float32)
    a_s2 = jnp.dot(h2, a2s_ref[...], preferred_element_type=jnp.float32)
    a_d2 = jnp.dot(h2, a2d_ref[...], preferred_element_type=jnp.float32)
    pad15 = jnp.zeros((N_NODES, 15), jnp.float32)
    t2s_ref[...] = jnp.concatenate([h2, a_s2, pad15], axis=1)
    t2d_ref[...] = jnp.concatenate(
        [jnp.concatenate([a_d2, pad15], axis=1),
         jnp.zeros((N_PAD - N_NODES, 16), jnp.float32)], axis=0)


def _tc_final_body(p0_ref, p1_ref, t2s_ref, t2d_ref, b2_ref, out_ref):
    P = p0_ref[:N_NODES] + p1_ref[:N_NODES]
    msg = P[:, :N_CLASSES]
    den = P[:, N_CLASSES:N_CLASSES + 1]
    h2 = t2s_ref[:, :N_CLASSES]
    as2 = t2s_ref[:, N_CLASSES:N_CLASSES + 1]
    ad2 = t2d_ref[:N_NODES, 0:1]
    wself = jnp.exp(_leaky(as2 + ad2))
    out_ref[...] = (msg + wself * h2) / (den + wself + 1e-16) + b2_ref[...]


_MESH = plsc.VectorSubcoreMesh(
    core_axis_name="c", subcore_axis_name="s", num_cores=NC, num_subcores=NS)

_SC_PARAMS = pltpu.CompilerParams(
    use_tc_tiling_on_sc=False, needs_layout_passes=False)


def _sc_pipeline(compute, tsrc, tdst, ei_hbm, out_hbm,
                 ibuf, rows, adrows, acc, scat, zbuf, sgs, sss, sis, fwidth):
    c = lax.axis_index("c")
    s = lax.axis_index("s")
    wid = c * NS + s
    base = wid * E_PER_W

    def idx_cp(g, bi):
        off = base + g * CHUNK
        return pltpu.make_async_copy(
            ei_hbm.at[pl.ds(0, 2), pl.ds(off, CHUNK)], ibuf.at[bi], sis[bi])

    def gather_cps(b, bi):
        return (pltpu.make_async_copy(tsrc.at[ibuf.at[bi, 0]], rows.at[b], sgs[b]),
                pltpu.make_async_copy(tdst.at[ibuf.at[bi, 1]], adrows.at[b], sgs[b]))

    def scatter_cp(b, bi):
        return pltpu.make_async_copy(scat(b), acc.at[ibuf.at[bi, 1]],
                                     sss[b % len(sss)])

    cp0 = idx_cp(0, 0)
    cp0.start()
    idx_cp(1, 1).start()

    zero16 = jnp.zeros((16,), jnp.float32)
    row0 = s * ROWS_PER_TILE

    def zrow(r, carry):
        for j in range(fwidth // 16):
            zbuf[r, pl.ds(16 * j, 16)] = zero16
        return carry

    lax.fori_loop(0, CHUNK, zrow, 0, unroll=4)
    zsems = sgs + sss
    zcps = [pltpu.make_async_copy(
        zbuf, acc.at[pl.ds(row0 + k * CHUNK, CHUNK)],
        zsems[k % len(zsems)]) for k in range(ROWS_PER_TILE // CHUNK)]
    for cp in zcps:
        cp.start()
    for cp in zcps:
        cp.wait()
    plsc.subcore_barrier()

    cp0.wait()
    for cp in gather_cps(0, 0):
        cp.start()

    def body(t, carry):
        for u in range(UNROLL):
            g = t * UNROLL + u
            b = u % NRB
            bn = (u + 1) % NRB

            @pl.when(g >= 1)
            def _():
                scatter_cp(bn, (u + 3) % NIB).wait()

            @pl.when(g < N_CHUNKS - 2)
            def _():
                idx_cp(g + 2, (u + 2) % NIB).start()

            idx_cp(g + 1, (u + 1) % NIB).wait()
            for cp in gather_cps(bn, (u + 1) % NIB):
                cp.start()
            for cp in gather_cps(b, u):
                cp.wait()
            compute(b, u)
            scatter_cp(b, u).start(add=True)
        return carry

    lax.fori_loop(0, (N_CHUNKS - 1) // UNROLL, body, 0)

    scatter_cp(1, 3).wait()
    for cp in gather_cps(0, 0):
        cp.wait()
    compute(0, 0)
    cp_last = scatter_cp(0, 0)
    cp_last.start(add=True)
    cp_last.wait()

    plsc.subcore_barrier()
    pltpu.sync_copy(acc.at[pl.ds(s * ROWS_PER_TILE, ROWS_PER_TILE)],
                    out_hbm.at[c, pl.ds(s * ROWS_PER_TILE, ROWS_PER_TILE)])


def _sc_edges1(tsrc_hbm, tdst_hbm, ei_hbm, out_hbm,
               ibuf, rowsp, rowsf, adrows, acc,
               sg0, sg1, ss0, si0, si1, si2, si3):
    mask_hi = jnp.full((16,), -65536, jnp.int32)

    def unpack_lo(v):
        return lax.bitcast_convert_type(v << 16, jnp.float32)

    def unpack_hi(v):
        return lax.bitcast_convert_type(v & mask_hi, jnp.float32)

    def compute(b, bi):
        def edge(e, c2):
            as16 = unpack_lo(rowsp[b, e, pl.ds(64, 16)])
            ad16 = adrows[b, e, :]
            w = jnp.exp(_leaky(as16 + ad16))
            rowsf[e, pl.ds(N_FEAT, 16)] = w
            for j in range(4):
                v = rowsp[b, e, pl.ds(16 * j, 16)]
                rowsf[e, pl.ds(32 * j, 16)] = unpack_lo(v) * w[2 * j]
                rowsf[e, pl.ds(32 * j + 16, 16)] = unpack_hi(v) * w[2 * j + 1]
            return c2

        lax.fori_loop(0, CHUNK, edge, 0, unroll=4)

    _sc_pipeline(compute, tsrc_hbm, tdst_hbm, ei_hbm, out_hbm,
                 ibuf, rowsp, adrows, acc, lambda b: rowsf, rowsf,
                 [sg0, sg1], [ss0], [si0, si1, si2, si3], F1)


_sc_edges1_call = functools.partial(
    pl.kernel,
    out_type=jax.ShapeDtypeStruct((NC, N_PAD, F1), jnp.float32),
    mesh=_MESH,
    compiler_params=_SC_PARAMS,
    scratch_types=[
        pltpu.VMEM((NIB, 2, CHUNK), jnp.int32),
        pltpu.VMEM((NRB, CHUNK, F1P), jnp.int32),
        pltpu.VMEM((CHUNK, F1), jnp.float32),
        pltpu.VMEM((NRB, CHUNK, 16), jnp.float32),
        pltpu.VMEM_SHARED((N_PAD, F1), jnp.float32),
    ] + [pltpu.SemaphoreType.DMA] * (NRB + 1 + NIB),
)(_sc_edges1)


def _sc_edges2(tsrc_hbm, tdst_hbm, ei_hbm, out_hbm,
               ibuf, rows, adrows, acc,
               sg0, sg1, ss0, ss1, si0, si1, si2, si3):
    lanes = lax.iota(jnp.int32, 16)

    def compute(b, bi):
        def group(q, c2):
            rb = q * 16
            ridx = rb + lanes
            a_s = plsc.load_gather(
                rows.at[b], [ridx, jnp.full((16,), N_CLASSES, jnp.int32)])
            a_d = plsc.load_gather(
                adrows.at[b], [ridx, jnp.zeros((16,), jnp.int32)])
            w = jnp.exp(_leaky(a_s + a_d))
            for e in range(16):
                rows[b, rb + e, pl.ds(0, 16)] = rows[b, rb + e, pl.ds(0, 16)] * w[e]
                rows[b, rb + e, pl.ds(16, 16)] = jnp.where(lanes == 0, w[e], 0.0)
            return c2

        lax.fori_loop(0, CHUNK // 16, group, 0, unroll=2)

    _sc_pipeline(compute, tsrc_hbm, tdst_hbm, ei_hbm, out_hbm,
                 ibuf, rows, adrows, acc, lambda b: rows.at[b], rows.at[0],
                 [sg0, sg1], [ss0, ss1], [si0, si1, si2, si3], F2)


_sc_edges2_call = functools.partial(
    pl.kernel,
    out_type=jax.ShapeDtypeStruct((NC, N_PAD, F2), jnp.float32),
    mesh=_MESH,
    compiler_params=_SC_PARAMS,
    scratch_types=[
        pltpu.VMEM((NIB, 2, CHUNK), jnp.int32),
        pltpu.VMEM((NRB, CHUNK, F2), jnp.float32),
        pltpu.VMEM((NRB, CHUNK, 16), jnp.float32),
        pltpu.VMEM_SHARED((N_PAD, F2), jnp.float32),
    ] + [pltpu.SemaphoreType.DMA] * (2 * NRB + NIB),
)(_sc_edges2)


def kernel(x, edge_index, W1, att_src1, att_dst1, b1, W2, att_src2, att_dst2, b2):
    f32 = jnp.float32

    lane = jnp.arange(N_FEAT)
    head_of_lane = lane // N_HID
    onehot = (head_of_lane[:, None] == jnp.arange(N_HEADS)[None, :]).astype(f32)
    asrc_m = att_src1.reshape(N_FEAT)[:, None] * onehot
    adst_m = att_dst1.reshape(N_FEAT)[:, None] * onehot
    rep = onehot.T
    a2s = att_src2.reshape(N_CLASSES, 1)
    a2d = att_dst2.reshape(N_CLASSES, 1)

    t1s, t1p, t1d = pl.pallas_call(
        _tc_prep_body,
        out_shape=[
            jax.ShapeDtypeStruct((N_NODES, F1), f32),
            jax.ShapeDtypeStruct((N_NODES, F1P), jnp.int32),
            jax.ShapeDtypeStruct((N_PAD, 16), f32),
        ],
    )(x, W1, asrc_m, adst_m)

    acc1 = _sc_edges1_call(t1p, t1d, edge_index)

    t2s, t2d = pl.pallas_call(
        _tc_combine1_body,
        out_shape=[
            jax.ShapeDtypeStruct((N_NODES, F2), f32),
            jax.ShapeDtypeStruct((N_PAD, 16), f32),
        ],
    )(acc1[0], acc1[1], t1s, t1d, b1, W2, a2s, a2d, rep)

    acc2 = _sc_edges2_call(t2s, t2d, edge_index)

    out = pl.pallas_call(
        _tc_final_body,
        out_shape=jax.ShapeDtypeStruct((N_NODES, N_CLASSES), f32),
    )(acc2[0], acc2[1], t2s, t2d, b2)

    return out


# speedup vs baseline: 1.3356x; 1.3356x over previous
import functools

import jax
import jax.numpy as jnp
from jax import lax
from jax.experimental import pallas as pl
from jax.experimental.pallas import tpu as pltpu
import jax.experimental.pallas.tpu_sc as plsc

N_NODES = 10000
N_FEAT = 128
N_HID = 16
N_HEADS = 8
N_CLASSES = 16
N_EDGES = 320000

NC = 2
NS = 16
NW = NC * NS
E_PER_W = N_EDGES // NW
CHUNK = 80
N_CHUNKS = E_PER_W // CHUNK
N_PAD = 10240
ROWS_PER_TILE = N_PAD // NS

F1 = 144
F2 = 32

NRB = 2
NIB = 4
UNROLL = 4


def _leaky(a):
    return jnp.where(a >= 0, a, 0.2 * a)


def _tc_prep_body(x_ref, w1_ref, asrc_ref, adst_ref, t1s_ref, t1d_ref):
    h = jnp.dot(x_ref[...], w1_ref[...], preferred_element_type=jnp.float32)
    a_s = jnp.dot(h, asrc_ref[...], preferred_element_type=jnp.float32)
    a_d = jnp.dot(h, adst_ref[...], preferred_element_type=jnp.float32)
    pad8 = jnp.zeros((N_NODES, 8), jnp.float32)
    t1s_ref[...] = jnp.concatenate([h, a_s, pad8], axis=1)
    t1d_ref[...] = jnp.concatenate(
        [jnp.concatenate([a_d, pad8], axis=1),
         jnp.zeros((N_PAD - N_NODES, 16), jnp.float32)], axis=0)


def _tc_combine1_body(p0_ref, p1_ref, t1s_ref, t1d_ref, b1_ref, w2_ref,
                      a2s_ref, a2d_ref, rep_ref, t2s_ref, t2d_ref):
    P = p0_ref[:N_NODES] + p1_ref[:N_NODES]
    msg = P[:, :N_FEAT]
    den = P[:, N_FEAT:N_FEAT + N_HEADS]
    h1 = t1s_ref[:, :N_FEAT]
    as1 = t1s_ref[:, N_FEAT:N_FEAT + N_HEADS]
    ad1 = t1d_ref[:N_NODES, :N_HEADS]
    wself = jnp.exp(_leaky(as1 + ad1))
    rep = rep_ref[...]
    num = msg + jnp.dot(wself, rep, preferred_element_type=jnp.float32) * h1
    den_w = jnp.dot(den + wself, rep, preferred_element_type=jnp.float32)
    out1 = num / (den_w + 1e-16) + b1_ref[...]
    hidden = jnp.where(out1 > 0, out1, jnp.exp(out1) - 1.0)
    h2 = jnp.dot(hidden, w2_ref[...], preferred_element_type=jnp.float32)
    a_s2 = jnp.dot(h2, a2s_ref[...], preferred_element_type=jnp.float32)
    a_d2 = jnp.dot(h2, a2d_ref[...], preferred_element_type=jnp.float32)
    pad15 = jnp.zeros((N_NODES, 15), jnp.float32)
    t2s_ref[...] = jnp.concatenate([h2, a_s2, pad15], axis=1)
    t2d_ref[...] = jnp.concatenate(
        [jnp.concatenate([a_d2, pad15], axis=1),
         jnp.zeros((N_PAD - N_NODES, 16), jnp.float32)], axis=0)


def _tc_final_body(p0_ref, p1_ref, t2s_ref, t2d_ref, b2_ref, out_ref):
    P = p0_ref[:N_NODES] + p1_ref[:N_NODES]
    msg = P[:, :N_CLASSES]
    den = P[:, N_CLASSES:N_CLASSES + 1]
    h2 = t2s_ref[:, :N_CLASSES]
    as2 = t2s_ref[:, N_CLASSES:N_CLASSES + 1]
    ad2 = t2d_ref[:N_NODES, 0:1]
    wself = jnp.exp(_leaky(as2 + ad2))
    out_ref[...] = (msg + wself * h2) / (den + wself + 1e-16) + b2_ref[...]


_MESH = plsc.VectorSubcoreMesh(
    core_axis_name="c", subcore_axis_name="s", num_cores=NC, num_subcores=NS)

_SC_PARAMS = pltpu.CompilerParams(
    use_tc_tiling_on_sc=False, needs_layout_passes=False)


def _sc_pipeline(compute, tsrc, tdst, ei_hbm, out_hbm,
                 ibuf, rows, adrows, acc, scat, zbuf, sgs, sss, sis, fwidth):
    c = lax.axis_index("c")
    s = lax.axis_index("s")
    wid = c * NS + s
    base = wid * E_PER_W

    def idx_cp(g, bi):
        off = base + g * CHUNK
        return pltpu.make_async_copy(
            ei_hbm.at[pl.ds(0, 2), pl.ds(off, CHUNK)], ibuf.at[bi], sis[bi])

    def gather_cps(b, bi):
        return (pltpu.make_async_copy(tsrc.at[ibuf.at[bi, 0]], rows.at[b], sgs[b]),
                pltpu.make_async_copy(tdst.at[ibuf.at[bi, 1]], adrows.at[b], sgs[b]))

    def scatter_cp(b, bi):
        return pltpu.make_async_copy(scat(b), acc.at[ibuf.at[bi, 1]],
                                     sss[b % len(sss)])

    cp0 = idx_cp(0, 0)
    cp0.start()
    idx_cp(1, 1).start()

    zero16 = jnp.zeros((16,), jnp.float32)
    row0 = s * ROWS_PER_TILE

    def zrow(r, carry):
        for j in range(fwidth // 16):
            zbuf[r, pl.ds(16 * j, 16)] = zero16
        return carry

    lax.fori_loop(0, CHUNK, zrow, 0, unroll=4)
    zsems = sgs + sss
    zcps = [pltpu.make_async_copy(
        zbuf, acc.at[pl.ds(row0 + k * CHUNK, CHUNK)],
        zsems[k % len(zsems)]) for k in range(ROWS_PER_TILE // CHUNK)]
    for cp in zcps:
        cp.start()
    for cp in zcps:
        cp.wait()
    plsc.subcore_barrier()

    cp0.wait()
    for cp in gather_cps(0, 0):
        cp.start()

    def body(t, carry):
        for u in range(UNROLL):
            g = t * UNROLL + u
            b = u % NRB
            bn = (u + 1) % NRB

            @pl.when(g >= 1)
            def _():
                scatter_cp(bn, (u + 3) % NIB).wait()

            @pl.when(g < N_CHUNKS - 2)
            def _():
                idx_cp(g + 2, (u + 2) % NIB).start()

            idx_cp(g + 1, (u + 1) % NIB).wait()
            for cp in gather_cps(bn, (u + 1) % NIB):
                cp.start()
            for cp in gather_cps(b, u):
                cp.wait()
            compute(b, u)
            scatter_cp(b, u).start(add=True)
        return carry

    lax.fori_loop(0, (N_CHUNKS - 1) // UNROLL, body, 0)

    scatter_cp(1, 3).wait()
    for cp in gather_cps(0, 0):
        cp.wait()
    compute(0, 0)
    cp_last = scatter_cp(0, 0)
    cp_last.start(add=True)
    cp_last.wait()

    plsc.subcore_barrier()
    pltpu.sync_copy(acc.at[pl.ds(s * ROWS_PER_TILE, ROWS_PER_TILE)],
                    out_hbm.at[c, pl.ds(s * ROWS_PER_TILE, ROWS_PER_TILE)])


def _sc_edges1(tsrc_hbm, tdst_hbm, ei_hbm, out_hbm,
               ibuf, rows, adrows, acc,
               sg0, sg1, ss0, ss1, si0, si1, si2, si3):
    def compute(b, bi):
        def edge(e, c2):
            as16 = rows[b, e, pl.ds(N_FEAT, 16)]
            ad16 = adrows[b, e, :]
            w = jnp.exp(_leaky(as16 + ad16))
            rows[b, e, pl.ds(N_FEAT, 16)] = w
            for h in range(N_HEADS):
                rows[b, e, pl.ds(h * 16, 16)] = rows[b, e, pl.ds(h * 16, 16)] * w[h]
            return c2

        lax.fori_loop(0, CHUNK, edge, 0, unroll=4)

    _sc_pipeline(compute, tsrc_hbm, tdst_hbm, ei_hbm, out_hbm,
                 ibuf, rows, adrows, acc, lambda b: rows.at[b], rows.at[0],
                 [sg0, sg1], [ss0, ss1], [si0, si1, si2, si3], F1)


_sc_edges1_call = functools.partial(
    pl.kernel,
    out_type=jax.ShapeDtypeStruct((NC, N_PAD, F1), jnp.float32),
    mesh=_MESH,
    compiler_params=_SC_PARAMS,
    scratch_types=[
        pltpu.VMEM((NIB, 2, CHUNK), jnp.int32),
        pltpu.VMEM((NRB, CHUNK, F1), jnp.float32),
        pltpu.VMEM((NRB, CHUNK, 16), jnp.float32),
        pltpu.VMEM_SHARED((N_PAD, F1), jnp.float32),
    ] + [pltpu.SemaphoreType.DMA] * (2 * NRB + NIB),
)(_sc_edges1)


def _sc_edges2(tsrc_hbm, tdst_hbm, ei_hbm, out_hbm,
               ibuf, rows, adrows, acc,
               sg0, sg1, ss0, ss1, si0, si1, si2, si3):
    lanes = lax.iota(jnp.int32, 16)

    def compute(b, bi):
        def group(q, c2):
            rb = q * 16
            ridx = rb + lanes
            a_s = plsc.load_gather(
                rows.at[b], [ridx, jnp.full((16,), N_CLASSES, jnp.int32)])
            a_d = plsc.load_gather(
                adrows.at[b], [ridx, jnp.zeros((16,), jnp.int32)])
            w = jnp.exp(_leaky(a_s + a_d))
            for e in range(16):
                rows[b, rb + e, pl.ds(0, 16)] = rows[b, rb + e, pl.ds(0, 16)] * w[e]
                rows[b, rb + e, pl.ds(16, 16)] = jnp.where(lanes == 0, w[e], 0.0)
            return c2

        lax.fori_loop(0, CHUNK // 16, group, 0, unroll=2)

    _sc_pipeline(compute, tsrc_hbm, tdst_hbm, ei_hbm, out_hbm,
                 ibuf, rows, adrows, acc, lambda b: rows.at[b], rows.at[0],
                 [sg0, sg1], [ss0, ss1], [si0, si1, si2, si3], F2)


_sc_edges2_call = functools.partial(
    pl.kernel,
    out_type=jax.ShapeDtypeStruct((NC, N_PAD, F2), jnp.float32),
    mesh=_MESH,
    compiler_params=_SC_PARAMS,
    scratch_types=[
        pltpu.VMEM((NIB, 2, CHUNK), jnp.int32),
        pltpu.VMEM((NRB, CHUNK, F2), jnp.float32),
        pltpu.VMEM((NRB, CHUNK, 16), jnp.float32),
        pltpu.VMEM_SHARED((N_PAD, F2), jnp.float32),
    ] + [pltpu.SemaphoreType.DMA] * (2 * NRB + NIB),
)(_sc_edges2)


def kernel(x, edge_index, W1, att_src1, att_dst1, b1, W2, att_src2, att_dst2, b2):
    f32 = jnp.float32

    lane = jnp.arange(N_FEAT)
    head_of_lane = lane // N_HID
    onehot = (head_of_lane[:, None] == jnp.arange(N_HEADS)[None, :]).astype(f32)
    asrc_m = att_src1.reshape(N_FEAT)[:, None] * onehot
    adst_m = att_dst1.reshape(N_FEAT)[:, None] * onehot
    rep = onehot.T
    a2s = att_src2.reshape(N_CLASSES, 1)
    a2d = att_dst2.reshape(N_CLASSES, 1)

    t1s, t1d = pl.pallas_call(
        _tc_prep_body,
        out_shape=[
            jax.ShapeDtypeStruct((N_NODES, F1), f32),
            jax.ShapeDtypeStruct((N_PAD, 16), f32),
        ],
    )(x, W1, asrc_m, adst_m)

    acc1 = _sc_edges1_call(t1s, t1d, edge_index)

    t2s, t2d = pl.pallas_call(
        _tc_combine1_body,
        out_shape=[
            jax.ShapeDtypeStruct((N_NODES, F2), f32),
            jax.ShapeDtypeStruct((N_PAD, 16), f32),
        ],
    )(acc1[0], acc1[1], t1s, t1d, b1, W2, a2s, a2d, rep)

    acc2 = _sc_edges2_call(t2s, t2d, edge_index)

    out = pl.pallas_call(
        _tc_final_body,
        out_shape=jax.ShapeDtypeStruct((N_NODES, N_CLASSES), f32),
    )(acc2[0], acc2[1], t2s, t2d, b2)

    return out
